# baseline (speedup 1.0000x reference)
import jax
import jax.numpy as jnp
from jax import lax
from jax.experimental import pallas as pl
from jax.experimental.pallas import tpu as pltpu
from jax.experimental.pallas import tpu_sc as plsc

NC = 2
NS = 16
L = 16
NW = NC * NS

B = 16384
D = 32
V = 1000000
BPW = B // NW
HALF = BPW // 2
CHUNK = 128
NGROUP = HALF // L

TBLK = 15616
NMAIN = 64 * TBLK
NTAIL = V - NMAIN


def _xpose_body(x_hbm, o_ref, xa, xb, sa, sb):
    g = pl.program_id(0)
    ng = pl.num_programs(0)

    @pl.when(g == 0)
    def _():
        pltpu.make_async_copy(
            x_hbm.at[:, pl.ds(0, TBLK)], xa, sa
        ).start()

    @pl.when(jnp.logical_and(g + 1 < ng, g % 2 == 0))
    def _():
        pltpu.make_async_copy(
            x_hbm.at[:, pl.ds((g + 1) * TBLK, TBLK)], xb, sb
        ).start()

    @pl.when(jnp.logical_and(g + 1 < ng, g % 2 == 1))
    def _():
        pltpu.make_async_copy(
            x_hbm.at[:, pl.ds((g + 1) * TBLK, TBLK)], xa, sa
        ).start()

    @pl.when(g % 2 == 0)
    def _():
        pltpu.make_async_copy(
            x_hbm.at[:, pl.ds(g * TBLK, TBLK)], xa, sa
        ).wait()
        o_ref[...] = xa[...].T

    @pl.when(g % 2 == 1)
    def _():
        pltpu.make_async_copy(
            x_hbm.at[:, pl.ds(g * TBLK, TBLK)], xb, sb
        ).wait()
        o_ref[...] = xb[...].T


@jax.jit
def _to_row_major(table_t):
    return pl.pallas_call(
        _xpose_body,
        grid=(NMAIN // TBLK,),
        in_specs=[pl.BlockSpec(memory_space=pl.ANY)],
        out_specs=pl.BlockSpec((TBLK, D), lambda g: (g, 0)),
        out_shape=jax.ShapeDtypeStruct((NMAIN, D), jnp.float32),
        scratch_shapes=[
            pltpu.VMEM((D, TBLK), jnp.float32),
            pltpu.VMEM((D, TBLK), jnp.float32),
            pltpu.SemaphoreType.DMA,
            pltpu.SemaphoreType.DMA,
        ],
    )(table_t)


def _tail_body(x_ref, o_ref):
    o_ref[...] = x_ref[...].T


@jax.jit
def _tail_row_major(table_t):
    return pl.pallas_call(
        _tail_body,
        in_specs=[pl.BlockSpec((D, NTAIL), lambda: (0, 0))],
        out_specs=pl.BlockSpec((NTAIL, D), lambda: (0, 0)),
        out_shape=jax.ShapeDtypeStruct((NTAIL, D), jnp.float32),
    )(table_t[:, NMAIN:])


def _sc_kernel(idx6, ev_m, ed_m, at_m, ev_t, ed_t, at_t, out_hbm,
               idx_v, ph, pr, pt, nh, nr, nt, tv_ev, tv_ed, tv_at,
               out_v, sem):
    wid = lax.axis_index("s") * NC + lax.axis_index("c")
    base = wid * BPW

    pltpu.sync_copy(idx6.at[:, wid], idx_v.at[pl.ds(0, 6)])
    pltpu.sync_copy(ev_t, tv_ev)
    pltpu.sync_copy(ed_t, tv_ed)
    pltpu.sync_copy(at_t, tv_at)

    tables = (ev_m, ed_m, at_m, ev_m, ed_m, at_m)
    tails = (tv_ev, tv_ed, tv_at, tv_ev, tv_ed, tv_at)
    bufs = (ph, pr, pt, nh, nr, nt)

    lane = lax.iota(jnp.int32, L)
    zeros = jnp.zeros((L,), jnp.float32)
    ones = jnp.full((L,), 1.0, jnp.float32)
    s0 = pl.ds(0, L)
    s1 = pl.ds(L, L)

    nm = jnp.full((L,), NMAIN - 1, jnp.int32)

    for h in range(2):
        hb = h * HALF

        copies = []
        for j in range(6):
            def clamp_body(k, _):
                s = pl.ds(hb + k * L, L)
                idx_v[6 + j, s] = jnp.minimum(idx_v[j, s], nm)
                return 0
            lax.fori_loop(0, HALF // L, clamp_body, 0)

        for j in range(6):
            for c in range(2):
                cp = pltpu.make_async_copy(
                    tables[j].at[
                        idx_v.at[6 + j, pl.ds(hb + c * CHUNK, CHUNK)]
                    ],
                    bufs[j].at[pl.ds(c * CHUNK, CHUNK), :],
                    sem,
                )
                cp.start()
                copies.append(cp)
        for cp in copies:
            cp.wait()

        def margin_diff(b, raw):
            def val(j, sl):
                main = bufs[j][b, sl]
                tloc = jnp.maximum(raw[j] - NMAIN, 0)
                tail = tails[j][tloc, sl]
                return jnp.where(raw[j] >= NMAIN, tail, main)

            dp = jnp.abs(val(0, s0) + val(1, s0) - val(2, s0)) + jnp.abs(
                val(0, s1) + val(1, s1) - val(2, s1)
            )
            dn = jnp.abs(val(3, s0) + val(4, s0) - val(5, s0)) + jnp.abs(
                val(3, s1) + val(4, s1) - val(5, s1)
            )
            return jnp.sum(dp - dn)

        def group_body(g, _):
            rawv = [idx_v[j, pl.ds(hb + g * L, L)] for j in range(6)]
            vloss = zeros
            for u in range(L):
                sc = margin_diff(
                    g * L + u, [rawv[j][u] for j in range(6)]
                )
                vloss = jnp.where(
                    lane == u, lax.broadcast(sc, (L,)), vloss
                )
            out_v[pl.ds(hb + g * L, L)] = jnp.maximum(zeros, vloss + ones)
            return 0

        lax.fori_loop(0, NGROUP, group_body, 0)

    pltpu.sync_copy(out_v, out_hbm.at[pl.ds(base, BPW)])


@jax.jit
def _run(idx6, ev_m, ed_m, at_m, ev_t, ed_t, at_t):
    mesh = plsc.VectorSubcoreMesh(core_axis_name="c", subcore_axis_name="s")
    return pl.kernel(
        _sc_kernel,
        out_type=jax.ShapeDtypeStruct((B,), jnp.float32),
        mesh=mesh,
        compiler_params=pltpu.CompilerParams(
            needs_layout_passes=False, use_tc_tiling_on_sc=False
        ),
        scratch_types=[
            pltpu.VMEM((12, BPW), jnp.int32),
            pltpu.VMEM((HALF, D), jnp.float32),
            pltpu.VMEM((HALF, D), jnp.float32),
            pltpu.VMEM((HALF, D), jnp.float32),
            pltpu.VMEM((HALF, D), jnp.float32),
            pltpu.VMEM((HALF, D), jnp.float32),
            pltpu.VMEM((HALF, D), jnp.float32),
            pltpu.VMEM((NTAIL, D), jnp.float32),
            pltpu.VMEM((NTAIL, D), jnp.float32),
            pltpu.VMEM((NTAIL, D), jnp.float32),
            pltpu.VMEM((BPW,), jnp.float32),
            pltpu.SemaphoreType.DMA,
        ],
    )(idx6, ev_m, ed_m, at_m, ev_t, ed_t, at_t)


def kernel(pos_triplets, neg_triplets, event_em, edgetype_em, attrib_em):
    idx6 = jnp.concatenate(
        [pos_triplets.T, neg_triplets.T], axis=0
    ).reshape(6, NW, BPW)
    ev_tv, ed_tv, at_tv = event_em.T, edgetype_em.T, attrib_em.T
    return _run(
        idx6,
        _to_row_major(ev_tv), _to_row_major(ed_tv), _to_row_major(at_tv),
        _tail_row_major(ev_tv), _tail_row_major(ed_tv),
        _tail_row_major(at_tv),
    )
